# baseline (speedup 1.0000x reference)
import functools

import jax
import jax.numpy as jnp
from jax import lax
from jax.experimental import pallas as pl
from jax.experimental.pallas import tpu as pltpu
from jax.experimental.pallas import tpu_sc as plsc


def _make_sc_kernel(B, L, E, NC, NS):
    NW = NC * NS
    BLK = 128
    assert B == NW * BLK
    EO, ES = E // 8, 8
    UL = 2
    NBUF = 4
    NU = L // UL
    assert EO * ES == E and L % UL == 0 and NU % NBUF == 0

    mesh = plsc.VectorSubcoreMesh(core_axis_name="c", subcore_axis_name="s")

    @functools.partial(
        pl.kernel,
        out_type=jax.ShapeDtypeStruct((L, EO, NW, ES, BLK), jnp.float32),
        mesh=mesh,
        scratch_types=[
            pltpu.VMEM((L, E), jnp.float32),
            pltpu.VMEM((16, L), jnp.int32),
            pltpu.VMEM((L, BLK), jnp.int32),
            pltpu.VMEM((NBUF, UL * BLK, E), jnp.float32),
            pltpu.VMEM((NBUF, UL, EO, ES, BLK), jnp.float32),
            pltpu.SemaphoreType.DMA((NBUF,)),
            pltpu.SemaphoreType.DMA((NBUF,)),
        ],
        compiler_params=pltpu.CompilerParams(
            use_tc_tiling_on_sc=False, needs_layout_passes=False),
    )
    def emb(tok_hbm, idx_hbm, pos_hbm, out_hbm, pos_v, idxr_v, idxt_v,
            rows_v, outb_v, gsem, osem):
        wid = lax.axis_index("s") * NC + lax.axis_index("c")
        b0 = wid * BLK
        pltpu.sync_copy(pos_hbm, pos_v)
        lanes = lax.iota(jnp.int32, 16)

        for j in range(BLK // 16):
            pltpu.sync_copy(idx_hbm.at[pl.ds(b0 + j * 16, 16), :], idxr_v)

            @pl.loop(0, L)
            def _transpose_idx(l):
                lvec = lanes * 0 + l
                idxt_v[l, pl.ds(j * 16, 16)] = plsc.load_gather(
                    idxr_v, [lanes, lvec])

        def fire_unit_gathers(u, b):
            for k in range(UL):
                pltpu.async_copy(
                    tok_hbm.at[idxt_v.at[u * UL + k, :]],
                    rows_v.at[b, pl.ds(k * BLK, BLK), :],
                    gsem.at[b])

        for b in range(NBUF - 1):
            fire_unit_gathers(b, b)

        @pl.loop(0, NU // NBUF)
        def _ring(g):
            for b in range(NBUF):
                u = g * NBUF + b
                pltpu.make_async_copy(
                    tok_hbm.at[pl.ds(0, UL * BLK), :], rows_v.at[b], gsem.at[b]
                ).wait()

                @pl.when(g > 0)
                def _():
                    pltpu.make_async_copy(
                        outb_v.at[b], out_hbm.at[pl.ds(0, UL), :, 0, :, :],
                        osem.at[b]).wait()

                for k in range(UL):
                    l = u * UL + k
                    lvec = lanes * 0 + l

                    @plsc.parallel_loop(0, E, unroll=8)
                    def _emb_dim(e):
                        evec = lanes * 0 + e
                        pv = plsc.load_gather(pos_v, [lvec, evec])
                        eo, es = e // ES, e % ES
                        for j in range(BLK // 16):
                            v = plsc.load_gather(
                                rows_v.at[b],
                                [lanes + (k * BLK + j * 16), evec])
                            outb_v[b, k, eo, es, pl.ds(j * 16, 16)] = v + pv

                nxt = u + NBUF - 1
                if b == 0:
                    fire_unit_gathers(nxt, (b + NBUF - 1) % NBUF)
                else:
                    @pl.when(g < NU // NBUF - 1)
                    def _():
                        fire_unit_gathers(nxt, (b + NBUF - 1) % NBUF)

                pltpu.async_copy(
                    outb_v.at[b], out_hbm.at[pl.ds(u * UL, UL), :, wid, :, :],
                    osem.at[b])

        for b in range(NBUF):
            pltpu.make_async_copy(
                outb_v.at[b], out_hbm.at[pl.ds(0, UL), :, 0, :, :], osem.at[b]
            ).wait()

    return emb


def kernel(inputs, tok_table, pos_table):
    B, L = inputs.shape
    E = tok_table.shape[1]
    info = plsc.get_sparse_core_info()
    emb = _make_sc_kernel(B, L, E, info.num_cores, info.num_subcores)
    out5 = emb(tok_table, inputs.astype(jnp.int32), pos_table)
    return out5.transpose(2, 4, 0, 1, 3).reshape(B, L, E)


# speedup vs baseline: 1.5056x; 1.5056x over previous
import functools

import jax
import jax.numpy as jnp
from jax import lax
from jax.experimental import pallas as pl
from jax.experimental.pallas import tpu as pltpu
from jax.experimental.pallas import tpu_sc as plsc


def _make_sc_kernel(B, L, E, NC, NS):
    NW = NC * NS
    BLK = 128
    assert B == NW * BLK
    EO, ES = E // 8, 8
    UL = 2
    NBUF = 4
    NU = L // UL
    assert EO * ES == E and L % UL == 0 and NU % NBUF == 0

    mesh = plsc.VectorSubcoreMesh(core_axis_name="c", subcore_axis_name="s")

    @functools.partial(
        pl.kernel,
        out_type=jax.ShapeDtypeStruct((L, EO, NW, ES, BLK), jnp.float32),
        mesh=mesh,
        scratch_types=[
            pltpu.VMEM((L, E), jnp.float32),
            pltpu.VMEM((16, L), jnp.int32),
            pltpu.VMEM((L, BLK), jnp.int32),
            pltpu.VMEM((NBUF, UL * BLK, E), jnp.float32),
            pltpu.VMEM((UL * BLK, E + 1), jnp.float32),
            pltpu.VMEM((NBUF, UL, EO, ES, BLK), jnp.float32),
            pltpu.SemaphoreType.DMA((NBUF,)),
            pltpu.SemaphoreType.DMA((NBUF,)),
        ],
        compiler_params=pltpu.CompilerParams(
            use_tc_tiling_on_sc=False, needs_layout_passes=False),
    )
    def emb(tok_hbm, idx_hbm, pos_hbm, out_hbm, pos_v, idxr_v, idxt_v,
            rows_v, pad_v, outb_v, gsem, osem):
        wid = lax.axis_index("s") * NC + lax.axis_index("c")
        b0 = wid * BLK
        pltpu.sync_copy(pos_hbm, pos_v)
        lanes = lax.iota(jnp.int32, 16)

        for j in range(BLK // 16):
            pltpu.sync_copy(idx_hbm.at[pl.ds(b0 + j * 16, 16), :], idxr_v)

            @pl.loop(0, L)
            def _transpose_idx(l):
                lvec = lanes * 0 + l
                idxt_v[l, pl.ds(j * 16, 16)] = plsc.load_gather(
                    idxr_v, [lanes, lvec])

        def fire_unit_gathers(u, b):
            for k in range(UL):
                pltpu.async_copy(
                    tok_hbm.at[idxt_v.at[u * UL + k, :]],
                    rows_v.at[b, pl.ds(k * BLK, BLK), :],
                    gsem.at[b])

        for b in range(NBUF - 1):
            fire_unit_gathers(b, b)

        @pl.loop(0, NU // NBUF)
        def _ring(g):
            for b in range(NBUF):
                u = g * NBUF + b
                pltpu.make_async_copy(
                    tok_hbm.at[pl.ds(0, UL * BLK), :], rows_v.at[b], gsem.at[b]
                ).wait()

                for k in range(UL):
                    l = u * UL + k
                    p0 = pos_v[l, pl.ds(0, 16)]
                    p1 = pos_v[l, pl.ds(16, 16)]

                    @plsc.parallel_loop(0, BLK, unroll=4)
                    def _repitch(r):
                        pad_v[k * BLK + r, pl.ds(0, 16)] = (
                            rows_v[b, k * BLK + r, pl.ds(0, 16)] + p0)
                        pad_v[k * BLK + r, pl.ds(16, 16)] = (
                            rows_v[b, k * BLK + r, pl.ds(16, 16)] + p1)

                @pl.when(g > 0)
                def _():
                    pltpu.make_async_copy(
                        outb_v.at[b], out_hbm.at[pl.ds(0, UL), :, 0, :, :],
                        osem.at[b]).wait()

                for k in range(UL):
                    @plsc.parallel_loop(0, E, unroll=4)
                    def _emb_dim(e):
                        evec = lanes * 0 + e
                        eo, es = e // ES, e % ES
                        for j in range(BLK // 16):
                            v = plsc.load_gather(
                                pad_v, [lanes + (k * BLK + j * 16), evec])
                            outb_v[b, k, eo, es, pl.ds(j * 16, 16)] = v

                nxt = u + NBUF - 1
                if b == 0:
                    fire_unit_gathers(nxt, (b + NBUF - 1) % NBUF)
                else:
                    @pl.when(g < NU // NBUF - 1)
                    def _():
                        fire_unit_gathers(nxt, (b + NBUF - 1) % NBUF)

                pltpu.async_copy(
                    outb_v.at[b], out_hbm.at[pl.ds(u * UL, UL), :, wid, :, :],
                    osem.at[b])

        for b in range(NBUF):
            pltpu.make_async_copy(
                outb_v.at[b], out_hbm.at[pl.ds(0, UL), :, 0, :, :], osem.at[b]
            ).wait()

    return emb


def kernel(inputs, tok_table, pos_table):
    B, L = inputs.shape
    E = tok_table.shape[1]
    info = plsc.get_sparse_core_info()
    emb = _make_sc_kernel(B, L, E, info.num_cores, info.num_subcores)
    out5 = emb(tok_table, inputs.astype(jnp.int32), pos_table)
    return out5.transpose(2, 4, 0, 1, 3).reshape(B, L, E)
